# baseline (speedup 1.0000x reference)
import functools

import jax
import jax.numpy as jnp
from jax import lax
from jax.experimental import pallas as pl
from jax.experimental.pallas import tpu as pltpu
from jax.experimental.pallas import tpu_sc as plsc

N = 10000
E = 320000
D = 128
EPS = 1e-5
NC, NS = 2, 16
NW = NC * NS
CH = 128
NCHUNK = 80
TCH = NW * NCHUNK
NCA = 104
NCB = 2 * NCHUNK - NCA
NCMAX = max(NCA, NCB)
TCHP = TCH + NCMAX - min(NCA, NCB)
EPAD = TCHP * CH - E
ROWS_PER_TILE = 640
ACC_ROWS = NS * ROWS_PER_TILE

_MESH = plsc.VectorSubcoreMesh(
    core_axis_name="c", subcore_axis_name="s", num_cores=NC, num_subcores=NS)


def _fill_rows(buf, nrows, val):
    def fill(i, _):
        for k in range(D // 16):
            buf[i, pl.ds(k * 16, 16)] = jnp.full((16,), val, jnp.float32)
        return 0
    lax.fori_loop(0, nrows, fill, 0)


def _deg_body(dst_hbm, out_hbm, dstv, onesv, acc, sem):
    c = lax.axis_index("c")
    s = lax.axis_index("s")
    w = s * NC + c
    _fill_rows(onesv, CH, 0.0)
    for k in range(ROWS_PER_TILE // CH):
        pltpu.sync_copy(onesv, acc.at[pl.ds(s * ROWS_PER_TILE + k * CH, CH)])
    _fill_rows(onesv, CH, 1.0)
    pltpu.async_copy(dst_hbm.at[pl.ds(w * NCHUNK, NCHUNK)], dstv, sem).wait()
    plsc.subcore_barrier()

    def step(j, _):
        pltpu.sync_copy(onesv, acc.at[dstv.at[j]], add=True)
        return 0
    lax.fori_loop(0, NCHUNK, step, 0)
    plsc.subcore_barrier()
    pltpu.sync_copy(acc.at[pl.ds(s * ROWS_PER_TILE, ROWS_PER_TILE)],
                    out_hbm.at[c, pl.ds(s * ROWS_PER_TILE, ROWS_PER_TILE)])


_deg_call = pl.kernel(
    _deg_body,
    out_type=jax.ShapeDtypeStruct((NC, ACC_ROWS, D), jnp.float32),
    mesh=_MESH,
    scratch_types=[
        pltpu.VMEM((NCHUNK, CH), jnp.int32),
        pltpu.VMEM((CH, D), jnp.float32),
        pltpu.VMEM_SHARED((ACC_ROWS, D), jnp.float32),
        pltpu.SemaphoreType.DMA,
    ],
)


def _conv_body(hs_hbm, src_hbm, dst_hbm, out_hbm, srcv, dstv, rows, acc, sem):
    c = lax.axis_index("c")
    s = lax.axis_index("s")
    w = s * NC + c

    _fill_rows(rows, CH, 0.0)
    for k in range(ROWS_PER_TILE // CH):
        pltpu.sync_copy(rows, acc.at[pl.ds(s * ROWS_PER_TILE + k * CH, CH)])
    nch = jnp.where(c == 0, NCA, NCB)
    base = jnp.where(c == 0, s * NCA, NS * NCA + s * NCB)
    pltpu.async_copy(src_hbm.at[pl.ds(base, NCMAX)], srcv, sem).wait()
    pltpu.async_copy(dst_hbm.at[pl.ds(base, NCMAX)], dstv, sem).wait()
    plsc.subcore_barrier()

    def step(j, _):
        pltpu.async_copy(hs_hbm.at[srcv.at[j]], rows, sem).wait()
        pltpu.sync_copy(rows, acc.at[dstv.at[j]], add=True)
        return 0
    lax.fori_loop(0, nch, step, 0)
    plsc.subcore_barrier()
    pltpu.sync_copy(acc.at[pl.ds(s * ROWS_PER_TILE, ROWS_PER_TILE)],
                    out_hbm.at[c, pl.ds(s * ROWS_PER_TILE, ROWS_PER_TILE)])


_conv_call = pl.kernel(
    _conv_body,
    out_type=jax.ShapeDtypeStruct((NC, ACC_ROWS, D), jnp.float32),
    mesh=_MESH,
    scratch_types=[
        pltpu.VMEM((NCMAX, CH), jnp.int32),
        pltpu.VMEM((NCMAX, CH), jnp.int32),
        pltpu.VMEM((CH, D), jnp.float32),
        pltpu.VMEM_SHARED((ACC_ROWS, D), jnp.float32),
        pltpu.SemaphoreType.DMA,
    ],
)


BLK = 2000
GRID = N // BLK


def _dot(a, b):
    return jnp.dot(a, b, preferred_element_type=jnp.float32,
                   precision=lax.Precision.HIGHEST)


def _t0_body(x, degp, r4, wa1, ba1, wa2, ba2, wc1, wfm, bfm, wfa, bfa,
             xa_o, hs1_o, s1_o, dinv_o, mres_o, ares_o):
    xb = x[...]
    deg = degp[0, :, 0:1] + degp[1, :, 0:1] + 1.0
    dinv = lax.rsqrt(jnp.maximum(deg, 1.0))
    dinvb = jnp.broadcast_to(dinv, (BLK, D))
    att = jnp.maximum(_dot(xb, wa1[...]) + ba1[...], 0.0)
    l4 = _dot(att, wa2[...]) + ba2[...]
    e4 = jnp.exp(l4 - jnp.max(l4, axis=1, keepdims=True))
    w4 = e4 / jnp.sum(e4, axis=1, keepdims=True)
    xa = xb * _dot(w4, r4[...])
    h1 = _dot(xa, wc1[...])
    xa_o[...] = xa
    hs1_o[...] = h1 * dinvb
    s1_o[...] = dinvb * dinvb * h1
    dinv_o[...] = dinvb
    mres_o[...] = jnp.maximum(_dot(xb, wfm[...]) + bfm[...], 0.0)
    ares_o[...] = jnp.maximum(_dot(xb, wfa[...]) + bfa[...], 0.0)


def _row_spec(last=D):
    return pl.BlockSpec((BLK, last), lambda i: (i, 0))


def _full_spec(shape):
    nd = len(shape)
    return pl.BlockSpec(shape, lambda i, _n=nd: (0,) * _n)


_t0_call = pl.pallas_call(
    _t0_body,
    grid=(GRID,),
    in_specs=[
        _row_spec(),
        pl.BlockSpec((NC, BLK, D), lambda i: (0, i, 0)),
        _full_spec((4, D)),
        _full_spec((D, D)), _full_spec((1, D)),
        _full_spec((D, 4)), _full_spec((1, 4)),
        _full_spec((D, D)),
        _full_spec((D, D)), _full_spec((1, D)),
        _full_spec((D, D)), _full_spec((1, D)),
    ],
    out_specs=[_row_spec()] * 6,
    out_shape=[jax.ShapeDtypeStruct((N, D), jnp.float32)] * 6,
    compiler_params=pltpu.CompilerParams(dimension_semantics=("arbitrary",)),
)


def _p_body(part, sterm, dinvb, bc, y_o, st_o):
    i = pl.program_id(0)
    a = part[0] + part[1]
    y = dinvb[...] * a + sterm[...] + bc[...]
    y_o[...] = y
    nb = float(BLK)
    m_b = jnp.sum(y, axis=0, keepdims=True) * (1.0 / nb)
    d = y - m_b
    m2_b = jnp.sum(d * d, axis=0, keepdims=True)
    pad = jnp.zeros((6, D), jnp.float32)

    @pl.when(i == 0)
    def _():
        st_o[...] = jnp.concatenate([m_b, m2_b, pad], axis=0)

    @pl.when(i > 0)
    def _():
        n_prev = i.astype(jnp.float32) * nb
        n_new = n_prev + nb
        mean_prev = st_o[0:1, :]
        m2_prev = st_o[1:2, :]
        delta = m_b - mean_prev
        mean_new = mean_prev + delta * (nb / n_new)
        m2_new = m2_prev + m2_b + delta * delta * (n_prev * nb / n_new)
        st_o[...] = jnp.concatenate([mean_new, m2_new, pad], axis=0)


_p_call = pl.pallas_call(
    _p_body,
    grid=(GRID,),
    in_specs=[
        pl.BlockSpec((NC, BLK, D), lambda i: (0, i, 0)),
        _row_spec(), _row_spec(), _full_spec((1, D)),
    ],
    out_specs=[_row_spec(), _full_spec((8, D))],
    out_shape=[jax.ShapeDtypeStruct((N, D), jnp.float32),
               jax.ShapeDtypeStruct((8, D), jnp.float32)],
    compiler_params=pltpu.CompilerParams(dimension_semantics=("arbitrary",)),
)


def _bn_from_stats(y, st, g, be, relu):
    m = st[0:1, :]
    v = st[1:2, :] * (1.0 / N)
    out = (y - m) * lax.rsqrt(v + EPS) * g + be
    if relu:
        out = jnp.maximum(out, 0.0)
    return out


def _q1_body(y, st, xa, dinvb, g, be, wc2, xc1_o, hs2_o, s2_o):
    x1 = _bn_from_stats(y[...], st[...], g[...], be[...], True)
    xc1 = x1 + xa[...]
    h2 = _dot(xc1, wc2[...])
    dv = dinvb[...]
    xc1_o[...] = xc1
    hs2_o[...] = h2 * dv
    s2_o[...] = dv * dv * h2


_q1_call = pl.pallas_call(
    _q1_body,
    grid=(GRID,),
    in_specs=[_row_spec(), _full_spec((8, D)), _row_spec(), _row_spec(),
              _full_spec((1, D)), _full_spec((1, D)), _full_spec((D, D))],
    out_specs=[_row_spec()] * 3,
    out_shape=[jax.ShapeDtypeStruct((N, D), jnp.float32)] * 3,
    compiler_params=pltpu.CompilerParams(dimension_semantics=("arbitrary",)),
)


def _q2_body(y, st, xc1, mres, dinvb, g, be, wgh, wgm, bgm, wc3,
             hs3_o, s3_o):
    x2 = _bn_from_stats(y[...], st[...], g[...], be[...], True)
    h = x2 + xc1[...]
    mr = mres[...]
    gm = jax.nn.sigmoid(_dot(h, wgh[...]) + _dot(mr, wgm[...]) + bgm[...])
    hg = gm * h + (1.0 - gm) * mr
    h3 = _dot(hg, wc3[...])
    dv = dinvb[...]
    hs3_o[...] = h3 * dv
    s3_o[...] = dv * dv * h3


_q2_call = pl.pallas_call(
    _q2_body,
    grid=(GRID,),
    in_specs=[_row_spec(), _full_spec((8, D)), _row_spec(), _row_spec(),
              _row_spec(), _full_spec((1, D)), _full_spec((1, D)),
              _full_spec((D, 1)), _full_spec((D, 1)), _full_spec((1, 1)),
              _full_spec((D, D))],
    out_specs=[_row_spec()] * 2,
    out_shape=[jax.ShapeDtypeStruct((N, D), jnp.float32)] * 2,
    compiler_params=pltpu.CompilerParams(dimension_semantics=("arbitrary",)),
)


def _q3_body(y, st, ares, g, be, wgh, wga, bga, wcl1, bcl1, wcl2, bcl2,
             out_o):
    hb = _bn_from_stats(y[...], st[...], g[...], be[...], False)
    ar = ares[...]
    ga = jax.nn.sigmoid(_dot(hb, wgh[...]) + _dot(ar, wga[...]) + bga[...])
    h = ga * hb + (1.0 - ga) * ar
    t = jnp.maximum(_dot(h, wcl1[...]) + bcl1[...], 0.0)
    out_o[...] = _dot(t, wcl2[...]) + bcl2[...]


_q3_call = pl.pallas_call(
    _q3_body,
    grid=(GRID,),
    in_specs=[_row_spec(), _full_spec((8, D)), _row_spec(),
              _full_spec((1, D)), _full_spec((1, D)),
              _full_spec((D, 1)), _full_spec((D, 1)), _full_spec((1, 1)),
              _full_spec((D, 64)), _full_spec((1, 64)),
              _full_spec((64, 1)), _full_spec((1, 1))],
    out_specs=[_row_spec(1)],
    out_shape=[jax.ShapeDtypeStruct((N, 1), jnp.float32)],
    compiler_params=pltpu.CompilerParams(dimension_semantics=("arbitrary",)),
)


def kernel(x, edge_index, params):
    p = params
    src = edge_index[0].astype(jnp.int32)
    dst = edge_index[1].astype(jnp.int32)
    src3 = jnp.concatenate([src, jnp.zeros((EPAD,), jnp.int32)])
    src3 = src3.reshape(TCHP, CH)
    dst3 = jnp.concatenate([dst, jnp.full((EPAD,), N, jnp.int32)])
    dst3 = dst3.reshape(TCHP, CH)

    r4 = (jnp.arange(D)[None, :] // 32 == jnp.arange(4)[:, None])
    r4 = r4.astype(jnp.float32)
    b2 = lambda v: v[None, :]

    degp = _deg_call(dst3)
    xa, hs1, s1, dinvb, mres, ares = _t0_call(
        x, degp, r4, p['Wa1'], b2(p['ba1']), p['Wa2'], b2(p['ba2']),
        p['Wc1'], p['Wfm'], b2(p['bfm']), p['Wfa'], b2(p['bfa']))

    part1 = _conv_call(hs1, src3, dst3)
    y1, st1 = _p_call(part1, s1, dinvb, b2(p['bc1']))
    xc1, hs2, s2 = _q1_call(y1, st1, xa, dinvb, b2(p['g1']), b2(p['be1']),
                            p['Wc2'])

    part2 = _conv_call(hs2, src3, dst3)
    y2, st2 = _p_call(part2, s2, dinvb, b2(p['bc2']))
    hs3, s3 = _q2_call(y2, st2, xc1, mres, dinvb, b2(p['g2']), b2(p['be2']),
                       p['Wgm'][:D], p['Wgm'][D:], b2(p['bgm']), p['Wc3'])

    part3 = _conv_call(hs3, src3, dst3)
    y3, st3 = _p_call(part3, s3, dinvb, b2(p['bc3']))
    (out,) = _q3_call(y3, st3, ares, b2(p['g3']), b2(p['be3']),
                      p['Wga'][:D], p['Wga'][D:], b2(p['bga']),
                      p['Wcl1'], b2(p['bcl1']), p['Wcl2'], b2(p['bcl2']))
    return out


# speedup vs baseline: 1.1706x; 1.1706x over previous
import functools

import jax
import jax.numpy as jnp
from jax import lax
from jax.experimental import pallas as pl
from jax.experimental.pallas import tpu as pltpu
from jax.experimental.pallas import tpu_sc as plsc

N = 10000
E = 320000
D = 128
EPS = 1e-5
NC, NS = 2, 16
NW = NC * NS
CH = 128
NCHUNK = 80
HALF = NCHUNK // 2
EPAD = NW * NCHUNK * CH - E
ROWS_PER_TILE = 640
ACC_ROWS = NS * ROWS_PER_TILE

_MESH = plsc.VectorSubcoreMesh(
    core_axis_name="c", subcore_axis_name="s", num_cores=NC, num_subcores=NS)


def _fill_rows(buf, nrows, val):
    def fill(i, _):
        for k in range(D // 16):
            buf[i, pl.ds(k * 16, 16)] = jnp.full((16,), val, jnp.float32)
        return 0
    lax.fori_loop(0, nrows, fill, 0)


def _deg_body(dst_hbm, out_hbm, dstv, onesv, acc, sem):
    c = lax.axis_index("c")
    s = lax.axis_index("s")
    w = s * NC + c
    _fill_rows(onesv, CH, 0.0)
    for k in range(ROWS_PER_TILE // CH):
        pltpu.sync_copy(onesv, acc.at[pl.ds(s * ROWS_PER_TILE + k * CH, CH)])
    _fill_rows(onesv, CH, 1.0)
    pltpu.async_copy(dst_hbm.at[w], dstv, sem).wait()
    plsc.subcore_barrier()

    def step(j, _):
        pltpu.sync_copy(onesv, acc.at[dstv.at[j]], add=True)
        return 0
    lax.fori_loop(0, NCHUNK, step, 0)
    plsc.subcore_barrier()
    pltpu.sync_copy(acc.at[pl.ds(s * ROWS_PER_TILE, ROWS_PER_TILE)],
                    out_hbm.at[c, pl.ds(s * ROWS_PER_TILE, ROWS_PER_TILE)])


_deg_call = pl.kernel(
    _deg_body,
    out_type=jax.ShapeDtypeStruct((NC, ACC_ROWS, D), jnp.float32),
    mesh=_MESH,
    scratch_types=[
        pltpu.VMEM((NCHUNK, CH), jnp.int32),
        pltpu.VMEM((CH, D), jnp.float32),
        pltpu.VMEM_SHARED((ACC_ROWS, D), jnp.float32),
        pltpu.SemaphoreType.DMA,
    ],
)


def _conv_body(hs_hbm, src_hbm, dst_hbm, out_hbm, srcv, dstv, rows0, rows1,
               acc, sem):
    c = lax.axis_index("c")
    s = lax.axis_index("s")
    w = s * NC + c

    _fill_rows(rows0, CH, 0.0)
    for k in range(ROWS_PER_TILE // CH):
        pltpu.sync_copy(rows0, acc.at[pl.ds(s * ROWS_PER_TILE + k * CH, CH)])
    plsc.subcore_barrier()

    for h in range(2):
        pltpu.async_copy(src_hbm.at[w, pl.ds(h * HALF, HALF)], srcv,
                         sem).wait()
        pltpu.async_copy(dst_hbm.at[w, pl.ds(h * HALF, HALF)], dstv,
                         sem).wait()

        def step(k, _):
            j = k * 2
            d0 = pltpu.async_copy(hs_hbm.at[srcv.at[j]], rows0, sem)
            d1 = pltpu.async_copy(hs_hbm.at[srcv.at[j + 1]], rows1, sem)
            d0.wait()
            pltpu.sync_copy(rows0, acc.at[dstv.at[j]], add=True)
            d1.wait()
            pltpu.sync_copy(rows1, acc.at[dstv.at[j + 1]], add=True)
            return 0
        lax.fori_loop(0, HALF // 2, step, 0)
    plsc.subcore_barrier()
    pltpu.sync_copy(acc.at[pl.ds(s * ROWS_PER_TILE, ROWS_PER_TILE)],
                    out_hbm.at[c, pl.ds(s * ROWS_PER_TILE, ROWS_PER_TILE)])


_conv_call = pl.kernel(
    _conv_body,
    out_type=jax.ShapeDtypeStruct((NC, ACC_ROWS, D), jnp.float32),
    mesh=_MESH,
    scratch_types=[
        pltpu.VMEM((HALF, CH), jnp.int32),
        pltpu.VMEM((HALF, CH), jnp.int32),
        pltpu.VMEM((CH, D), jnp.float32),
        pltpu.VMEM((CH, D), jnp.float32),
        pltpu.VMEM_SHARED((ACC_ROWS, D), jnp.float32),
        pltpu.SemaphoreType.DMA,
    ],
)


BLK = 2000
GRID = N // BLK


def _dot(a, b):
    return jnp.dot(a, b, preferred_element_type=jnp.float32,
                   precision=lax.Precision.HIGHEST)


def _t0_body(x, degp, r4, wa1, ba1, wa2, ba2, wc1, wfm, bfm, wfa, bfa,
             xa_o, hs1_o, s1_o, dinv_o, mres_o, ares_o):
    xb = x[...]
    deg = degp[0, :, 0:1] + degp[1, :, 0:1] + 1.0
    dinv = lax.rsqrt(jnp.maximum(deg, 1.0))
    dinvb = jnp.broadcast_to(dinv, (BLK, D))
    att = jnp.maximum(_dot(xb, wa1[...]) + ba1[...], 0.0)
    l4 = _dot(att, wa2[...]) + ba2[...]
    e4 = jnp.exp(l4 - jnp.max(l4, axis=1, keepdims=True))
    w4 = e4 / jnp.sum(e4, axis=1, keepdims=True)
    xa = xb * _dot(w4, r4[...])
    h1 = _dot(xa, wc1[...])
    xa_o[...] = xa
    hs1_o[...] = h1 * dinvb
    s1_o[...] = dinvb * dinvb * h1
    dinv_o[...] = dinvb
    mres_o[...] = jnp.maximum(_dot(xb, wfm[...]) + bfm[...], 0.0)
    ares_o[...] = jnp.maximum(_dot(xb, wfa[...]) + bfa[...], 0.0)


def _row_spec(last=D):
    return pl.BlockSpec((BLK, last), lambda i: (i, 0))


def _full_spec(shape):
    nd = len(shape)
    return pl.BlockSpec(shape, lambda i, _n=nd: (0,) * _n)


_t0_call = pl.pallas_call(
    _t0_body,
    grid=(GRID,),
    in_specs=[
        _row_spec(),
        pl.BlockSpec((NC, BLK, D), lambda i: (0, i, 0)),
        _full_spec((4, D)),
        _full_spec((D, D)), _full_spec((1, D)),
        _full_spec((D, 4)), _full_spec((1, 4)),
        _full_spec((D, D)),
        _full_spec((D, D)), _full_spec((1, D)),
        _full_spec((D, D)), _full_spec((1, D)),
    ],
    out_specs=[_row_spec()] * 6,
    out_shape=[jax.ShapeDtypeStruct((N, D), jnp.float32)] * 6,
    compiler_params=pltpu.CompilerParams(dimension_semantics=("arbitrary",)),
)


def _p_body(part, sterm, dinvb, bc, y_o, st_o):
    i = pl.program_id(0)
    a = part[0] + part[1]
    y = dinvb[...] * a + sterm[...] + bc[...]
    y_o[...] = y
    nb = float(BLK)
    m_b = jnp.sum(y, axis=0, keepdims=True) * (1.0 / nb)
    d = y - m_b
    m2_b = jnp.sum(d * d, axis=0, keepdims=True)
    pad = jnp.zeros((6, D), jnp.float32)

    @pl.when(i == 0)
    def _():
        st_o[...] = jnp.concatenate([m_b, m2_b, pad], axis=0)

    @pl.when(i > 0)
    def _():
        n_prev = i.astype(jnp.float32) * nb
        n_new = n_prev + nb
        mean_prev = st_o[0:1, :]
        m2_prev = st_o[1:2, :]
        delta = m_b - mean_prev
        mean_new = mean_prev + delta * (nb / n_new)
        m2_new = m2_prev + m2_b + delta * delta * (n_prev * nb / n_new)
        st_o[...] = jnp.concatenate([mean_new, m2_new, pad], axis=0)


_p_call = pl.pallas_call(
    _p_body,
    grid=(GRID,),
    in_specs=[
        pl.BlockSpec((NC, BLK, D), lambda i: (0, i, 0)),
        _row_spec(), _row_spec(), _full_spec((1, D)),
    ],
    out_specs=[_row_spec(), _full_spec((8, D))],
    out_shape=[jax.ShapeDtypeStruct((N, D), jnp.float32),
               jax.ShapeDtypeStruct((8, D), jnp.float32)],
    compiler_params=pltpu.CompilerParams(dimension_semantics=("arbitrary",)),
)


def _bn_from_stats(y, st, g, be, relu):
    m = st[0:1, :]
    v = st[1:2, :] * (1.0 / N)
    out = (y - m) * lax.rsqrt(v + EPS) * g + be
    if relu:
        out = jnp.maximum(out, 0.0)
    return out


def _q1_body(y, st, xa, dinvb, g, be, wc2, xc1_o, hs2_o, s2_o):
    x1 = _bn_from_stats(y[...], st[...], g[...], be[...], True)
    xc1 = x1 + xa[...]
    h2 = _dot(xc1, wc2[...])
    dv = dinvb[...]
    xc1_o[...] = xc1
    hs2_o[...] = h2 * dv
    s2_o[...] = dv * dv * h2


_q1_call = pl.pallas_call(
    _q1_body,
    grid=(GRID,),
    in_specs=[_row_spec(), _full_spec((8, D)), _row_spec(), _row_spec(),
              _full_spec((1, D)), _full_spec((1, D)), _full_spec((D, D))],
    out_specs=[_row_spec()] * 3,
    out_shape=[jax.ShapeDtypeStruct((N, D), jnp.float32)] * 3,
    compiler_params=pltpu.CompilerParams(dimension_semantics=("arbitrary",)),
)


def _q2_body(y, st, xc1, mres, dinvb, g, be, wgh, wgm, bgm, wc3,
             hs3_o, s3_o):
    x2 = _bn_from_stats(y[...], st[...], g[...], be[...], True)
    h = x2 + xc1[...]
    mr = mres[...]
    gm = jax.nn.sigmoid(_dot(h, wgh[...]) + _dot(mr, wgm[...]) + bgm[...])
    hg = gm * h + (1.0 - gm) * mr
    h3 = _dot(hg, wc3[...])
    dv = dinvb[...]
    hs3_o[...] = h3 * dv
    s3_o[...] = dv * dv * h3


_q2_call = pl.pallas_call(
    _q2_body,
    grid=(GRID,),
    in_specs=[_row_spec(), _full_spec((8, D)), _row_spec(), _row_spec(),
              _row_spec(), _full_spec((1, D)), _full_spec((1, D)),
              _full_spec((D, 1)), _full_spec((D, 1)), _full_spec((1, 1)),
              _full_spec((D, D))],
    out_specs=[_row_spec()] * 2,
    out_shape=[jax.ShapeDtypeStruct((N, D), jnp.float32)] * 2,
    compiler_params=pltpu.CompilerParams(dimension_semantics=("arbitrary",)),
)


def _q3_body(y, st, ares, g, be, wgh, wga, bga, wcl1, bcl1, wcl2, bcl2,
             out_o):
    hb = _bn_from_stats(y[...], st[...], g[...], be[...], False)
    ar = ares[...]
    ga = jax.nn.sigmoid(_dot(hb, wgh[...]) + _dot(ar, wga[...]) + bga[...])
    h = ga * hb + (1.0 - ga) * ar
    t = jnp.maximum(_dot(h, wcl1[...]) + bcl1[...], 0.0)
    out_o[...] = _dot(t, wcl2[...]) + bcl2[...]


_q3_call = pl.pallas_call(
    _q3_body,
    grid=(GRID,),
    in_specs=[_row_spec(), _full_spec((8, D)), _row_spec(),
              _full_spec((1, D)), _full_spec((1, D)),
              _full_spec((D, 1)), _full_spec((D, 1)), _full_spec((1, 1)),
              _full_spec((D, 64)), _full_spec((1, 64)),
              _full_spec((64, 1)), _full_spec((1, 1))],
    out_specs=[_row_spec(1)],
    out_shape=[jax.ShapeDtypeStruct((N, 1), jnp.float32)],
    compiler_params=pltpu.CompilerParams(dimension_semantics=("arbitrary",)),
)


def kernel(x, edge_index, params):
    p = params
    src = edge_index[0].astype(jnp.int32)
    dst = edge_index[1].astype(jnp.int32)
    src3 = jnp.concatenate([src, jnp.zeros((EPAD,), jnp.int32)])
    src3 = src3.reshape(NW, NCHUNK, CH)
    dst3 = jnp.concatenate([dst, jnp.full((EPAD,), N, jnp.int32)])
    dst3 = dst3.reshape(NW, NCHUNK, CH)

    r4 = (jnp.arange(D)[None, :] // 32 == jnp.arange(4)[:, None])
    r4 = r4.astype(jnp.float32)
    b2 = lambda v: v[None, :]

    degp = _deg_call(dst3)
    xa, hs1, s1, dinvb, mres, ares = _t0_call(
        x, degp, r4, p['Wa1'], b2(p['ba1']), p['Wa2'], b2(p['ba2']),
        p['Wc1'], p['Wfm'], b2(p['bfm']), p['Wfa'], b2(p['bfa']))

    part1 = _conv_call(hs1, src3, dst3)
    y1, st1 = _p_call(part1, s1, dinvb, b2(p['bc1']))
    xc1, hs2, s2 = _q1_call(y1, st1, xa, dinvb, b2(p['g1']), b2(p['be1']),
                            p['Wc2'])

    part2 = _conv_call(hs2, src3, dst3)
    y2, st2 = _p_call(part2, s2, dinvb, b2(p['bc2']))
    hs3, s3 = _q2_call(y2, st2, xc1, mres, dinvb, b2(p['g2']), b2(p['be2']),
                       p['Wgm'][:D], p['Wgm'][D:], b2(p['bgm']), p['Wc3'])

    part3 = _conv_call(hs3, src3, dst3)
    y3, st3 = _p_call(part3, s3, dinvb, b2(p['bc3']))
    (out,) = _q3_call(y3, st3, ares, b2(p['g3']), b2(p['be3']),
                      p['Wga'][:D], p['Wga'][D:], b2(p['bga']),
                      p['Wcl1'], b2(p['bcl1']), p['Wcl2'], b2(p['bcl2']))
    return out


# speedup vs baseline: 1.5362x; 1.3123x over previous
import functools

import jax
import jax.numpy as jnp
from jax import lax
from jax.experimental import pallas as pl
from jax.experimental.pallas import tpu as pltpu
from jax.experimental.pallas import tpu_sc as plsc

N = 10000
E = 320000
D = 128
EPS = 1e-5
NC, NS = 2, 16
NW = NC * NS
CH = 128
NCHUNK = 79
EPAD = NW * NCHUNK * CH - E
ROWS_PER_TILE = 640
ACC_ROWS = NS * ROWS_PER_TILE

_MESH = plsc.VectorSubcoreMesh(
    core_axis_name="c", subcore_axis_name="s", num_cores=NC, num_subcores=NS)


def _fill_rows(buf, nrows, val):
    def fill(i, _):
        for k in range(D // 16):
            buf[i, pl.ds(k * 16, 16)] = jnp.full((16,), val, jnp.float32)
        return 0
    lax.fori_loop(0, nrows, fill, 0)


def _deg_body(dst_hbm, out_hbm, dstv, onesv, acc, sem):
    c = lax.axis_index("c")
    s = lax.axis_index("s")
    w = s * NC + c
    _fill_rows(onesv, CH, 0.0)
    for k in range(ROWS_PER_TILE // CH):
        pltpu.sync_copy(onesv, acc.at[pl.ds(s * ROWS_PER_TILE + k * CH, CH)])
    _fill_rows(onesv, CH, 1.0)
    pltpu.async_copy(dst_hbm.at[w], dstv, sem).wait()
    plsc.subcore_barrier()

    def step(j, _):
        pltpu.sync_copy(onesv, acc.at[dstv.at[j]], add=True)
        return 0
    lax.fori_loop(0, NCHUNK, step, 0)
    plsc.subcore_barrier()
    pltpu.sync_copy(acc.at[pl.ds(s * ROWS_PER_TILE, ROWS_PER_TILE)],
                    out_hbm.at[c, pl.ds(s * ROWS_PER_TILE, ROWS_PER_TILE)])


_deg_call = pl.kernel(
    _deg_body,
    out_type=jax.ShapeDtypeStruct((NC, ACC_ROWS, D), jnp.float32),
    mesh=_MESH,
    scratch_types=[
        pltpu.VMEM((NCHUNK, CH), jnp.int32),
        pltpu.VMEM((CH, D), jnp.float32),
        pltpu.VMEM_SHARED((ACC_ROWS, D), jnp.float32),
        pltpu.SemaphoreType.DMA,
    ],
)


def _conv_body(hs_hbm, src_hbm, dst_hbm, out_hbm, srcv, dstv, rows, acc, sem):
    c = lax.axis_index("c")
    s = lax.axis_index("s")
    w = s * NC + c

    _fill_rows(rows, CH, 0.0)
    for k in range(ROWS_PER_TILE // CH):
        pltpu.sync_copy(rows, acc.at[pl.ds(s * ROWS_PER_TILE + k * CH, CH)])
    pltpu.async_copy(src_hbm.at[w], srcv, sem).wait()
    pltpu.async_copy(dst_hbm.at[w], dstv, sem).wait()
    plsc.subcore_barrier()

    def step(j, _):
        pltpu.async_copy(hs_hbm.at[srcv.at[j]], rows, sem).wait()
        pltpu.sync_copy(rows, acc.at[dstv.at[j]], add=True)
        return 0
    lax.fori_loop(0, NCHUNK, step, 0)
    plsc.subcore_barrier()
    pltpu.sync_copy(acc.at[pl.ds(s * ROWS_PER_TILE, ROWS_PER_TILE)],
                    out_hbm.at[c, pl.ds(s * ROWS_PER_TILE, ROWS_PER_TILE)])


_conv_call = pl.kernel(
    _conv_body,
    out_type=jax.ShapeDtypeStruct((NC, ACC_ROWS, D), jnp.float32),
    mesh=_MESH,
    scratch_types=[
        pltpu.VMEM((NCHUNK, CH), jnp.int32),
        pltpu.VMEM((NCHUNK, CH), jnp.int32),
        pltpu.VMEM((CH, D), jnp.float32),
        pltpu.VMEM_SHARED((ACC_ROWS, D), jnp.float32),
        pltpu.SemaphoreType.DMA,
    ],
)


BLK = 2000
GRID = N // BLK


def _dot(a, b):
    return jnp.dot(a, b, preferred_element_type=jnp.float32,
                   precision=lax.Precision.HIGHEST)


def _t0_body(x, degp, r4, wa1, ba1, wa2, ba2, wc1, wfm, bfm, wfa, bfa,
             xa_o, hs1_o, s1_o, dinv_o, mres_o, ares_o):
    xb = x[...]
    deg = degp[0, :, 0:1] + degp[1, :, 0:1] + 1.0
    dinv = lax.rsqrt(jnp.maximum(deg, 1.0))
    dinvb = jnp.broadcast_to(dinv, (BLK, D))
    att = jnp.maximum(_dot(xb, wa1[...]) + ba1[...], 0.0)
    l4 = _dot(att, wa2[...]) + ba2[...]
    e4 = jnp.exp(l4 - jnp.max(l4, axis=1, keepdims=True))
    w4 = e4 / jnp.sum(e4, axis=1, keepdims=True)
    xa = xb * _dot(w4, r4[...])
    h1 = _dot(xa, wc1[...])
    xa_o[...] = xa
    hs1_o[...] = h1 * dinvb
    s1_o[...] = dinvb * dinvb * h1
    dinv_o[...] = dinvb
    mres_o[...] = jnp.maximum(_dot(xb, wfm[...]) + bfm[...], 0.0)
    ares_o[...] = jnp.maximum(_dot(xb, wfa[...]) + bfa[...], 0.0)


def _row_spec(last=D):
    return pl.BlockSpec((BLK, last), lambda i: (i, 0))


def _full_spec(shape):
    nd = len(shape)
    return pl.BlockSpec(shape, lambda i, _n=nd: (0,) * _n)


_t0_call = pl.pallas_call(
    _t0_body,
    grid=(GRID,),
    in_specs=[
        _row_spec(),
        pl.BlockSpec((NC, BLK, D), lambda i: (0, i, 0)),
        _full_spec((4, D)),
        _full_spec((D, D)), _full_spec((1, D)),
        _full_spec((D, 4)), _full_spec((1, 4)),
        _full_spec((D, D)),
        _full_spec((D, D)), _full_spec((1, D)),
        _full_spec((D, D)), _full_spec((1, D)),
    ],
    out_specs=[_row_spec()] * 6,
    out_shape=[jax.ShapeDtypeStruct((N, D), jnp.float32)] * 6,
    compiler_params=pltpu.CompilerParams(dimension_semantics=("arbitrary",)),
)


def _p_body(part, sterm, dinvb, bc, y_o, st_o):
    i = pl.program_id(0)
    a = part[0] + part[1]
    y = dinvb[...] * a + sterm[...] + bc[...]
    y_o[...] = y
    nb = float(BLK)
    m_b = jnp.sum(y, axis=0, keepdims=True) * (1.0 / nb)
    d = y - m_b
    m2_b = jnp.sum(d * d, axis=0, keepdims=True)
    pad = jnp.zeros((6, D), jnp.float32)

    @pl.when(i == 0)
    def _():
        st_o[...] = jnp.concatenate([m_b, m2_b, pad], axis=0)

    @pl.when(i > 0)
    def _():
        n_prev = i.astype(jnp.float32) * nb
        n_new = n_prev + nb
        mean_prev = st_o[0:1, :]
        m2_prev = st_o[1:2, :]
        delta = m_b - mean_prev
        mean_new = mean_prev + delta * (nb / n_new)
        m2_new = m2_prev + m2_b + delta * delta * (n_prev * nb / n_new)
        st_o[...] = jnp.concatenate([mean_new, m2_new, pad], axis=0)


_p_call = pl.pallas_call(
    _p_body,
    grid=(GRID,),
    in_specs=[
        pl.BlockSpec((NC, BLK, D), lambda i: (0, i, 0)),
        _row_spec(), _row_spec(), _full_spec((1, D)),
    ],
    out_specs=[_row_spec(), _full_spec((8, D))],
    out_shape=[jax.ShapeDtypeStruct((N, D), jnp.float32),
               jax.ShapeDtypeStruct((8, D), jnp.float32)],
    compiler_params=pltpu.CompilerParams(dimension_semantics=("arbitrary",)),
)


def _bn_from_stats(y, st, g, be, relu):
    m = st[0:1, :]
    v = st[1:2, :] * (1.0 / N)
    out = (y - m) * lax.rsqrt(v + EPS) * g + be
    if relu:
        out = jnp.maximum(out, 0.0)
    return out


def _q1_body(y, st, xa, dinvb, g, be, wc2, xc1_o, hs2_o, s2_o):
    x1 = _bn_from_stats(y[...], st[...], g[...], be[...], True)
    xc1 = x1 + xa[...]
    h2 = _dot(xc1, wc2[...])
    dv = dinvb[...]
    xc1_o[...] = xc1
    hs2_o[...] = h2 * dv
    s2_o[...] = dv * dv * h2


_q1_call = pl.pallas_call(
    _q1_body,
    grid=(GRID,),
    in_specs=[_row_spec(), _full_spec((8, D)), _row_spec(), _row_spec(),
              _full_spec((1, D)), _full_spec((1, D)), _full_spec((D, D))],
    out_specs=[_row_spec()] * 3,
    out_shape=[jax.ShapeDtypeStruct((N, D), jnp.float32)] * 3,
    compiler_params=pltpu.CompilerParams(dimension_semantics=("arbitrary",)),
)


def _q2_body(y, st, xc1, mres, dinvb, g, be, wgh, wgm, bgm, wc3,
             hs3_o, s3_o):
    x2 = _bn_from_stats(y[...], st[...], g[...], be[...], True)
    h = x2 + xc1[...]
    mr = mres[...]
    gm = jax.nn.sigmoid(_dot(h, wgh[...]) + _dot(mr, wgm[...]) + bgm[...])
    hg = gm * h + (1.0 - gm) * mr
    h3 = _dot(hg, wc3[...])
    dv = dinvb[...]
    hs3_o[...] = h3 * dv
    s3_o[...] = dv * dv * h3


_q2_call = pl.pallas_call(
    _q2_body,
    grid=(GRID,),
    in_specs=[_row_spec(), _full_spec((8, D)), _row_spec(), _row_spec(),
              _row_spec(), _full_spec((1, D)), _full_spec((1, D)),
              _full_spec((D, 1)), _full_spec((D, 1)), _full_spec((1, 1)),
              _full_spec((D, D))],
    out_specs=[_row_spec()] * 2,
    out_shape=[jax.ShapeDtypeStruct((N, D), jnp.float32)] * 2,
    compiler_params=pltpu.CompilerParams(dimension_semantics=("arbitrary",)),
)


def _q3_body(y, st, ares, g, be, wgh, wga, bga, wcl1, bcl1, wcl2, bcl2,
             out_o):
    hb = _bn_from_stats(y[...], st[...], g[...], be[...], False)
    ar = ares[...]
    ga = jax.nn.sigmoid(_dot(hb, wgh[...]) + _dot(ar, wga[...]) + bga[...])
    h = ga * hb + (1.0 - ga) * ar
    t = jnp.maximum(_dot(h, wcl1[...]) + bcl1[...], 0.0)
    out_o[...] = _dot(t, wcl2[...]) + bcl2[...]


_q3_call = pl.pallas_call(
    _q3_body,
    grid=(GRID,),
    in_specs=[_row_spec(), _full_spec((8, D)), _row_spec(),
              _full_spec((1, D)), _full_spec((1, D)),
              _full_spec((D, 1)), _full_spec((D, 1)), _full_spec((1, 1)),
              _full_spec((D, 64)), _full_spec((1, 64)),
              _full_spec((64, 1)), _full_spec((1, 1))],
    out_specs=[_row_spec(1)],
    out_shape=[jax.ShapeDtypeStruct((N, 1), jnp.float32)],
    compiler_params=pltpu.CompilerParams(dimension_semantics=("arbitrary",)),
)


def kernel(x, edge_index, params):
    p = params
    src = edge_index[0].astype(jnp.int32)
    dst = edge_index[1].astype(jnp.int32)
    src3 = jnp.concatenate([src, jnp.zeros((EPAD,), jnp.int32)])
    src3 = src3.reshape(NW, NCHUNK, CH)
    dst3 = jnp.concatenate([dst, jnp.full((EPAD,), N, jnp.int32)])
    dst3 = dst3.reshape(NW, NCHUNK, CH)

    r4 = (jnp.arange(D)[None, :] // 32 == jnp.arange(4)[:, None])
    r4 = r4.astype(jnp.float32)
    b2 = lambda v: v[None, :]

    degp = _deg_call(dst3)
    xa, hs1, s1, dinvb, mres, ares = _t0_call(
        x, degp, r4, p['Wa1'], b2(p['ba1']), p['Wa2'], b2(p['ba2']),
        p['Wc1'], p['Wfm'], b2(p['bfm']), p['Wfa'], b2(p['bfa']))

    part1 = _conv_call(hs1, src3, dst3)
    y1, st1 = _p_call(part1, s1, dinvb, b2(p['bc1']))
    xc1, hs2, s2 = _q1_call(y1, st1, xa, dinvb, b2(p['g1']), b2(p['be1']),
                            p['Wc2'])

    part2 = _conv_call(hs2, src3, dst3)
    y2, st2 = _p_call(part2, s2, dinvb, b2(p['bc2']))
    hs3, s3 = _q2_call(y2, st2, xc1, mres, dinvb, b2(p['g2']), b2(p['be2']),
                       p['Wgm'][:D], p['Wgm'][D:], b2(p['bgm']), p['Wc3'])

    part3 = _conv_call(hs3, src3, dst3)
    y3, st3 = _p_call(part3, s3, dinvb, b2(p['bc3']))
    (out,) = _q3_call(y3, st3, ares, b2(p['g3']), b2(p['be3']),
                      p['Wga'][:D], p['Wga'][D:], b2(p['bga']),
                      p['Wcl1'], b2(p['bcl1']), p['Wcl2'], b2(p['bcl2']))
    return out
